# baseline (speedup 1.0000x reference)
import jax
import jax.numpy as jnp
from jax.experimental import pallas as pl


def _stats_body(s_ref, d_ref, a_ref, o_ref):
    i = pl.program_id(0)
    s = s_ref[...]
    d = d_ref[...]
    a = a_ref[...]
    ones = jnp.ones((s.shape[0], 1), jnp.float32)
    y = jnp.concatenate([s, d, a, ones], axis=1)
    g = jax.lax.dot_general(
        y, y, (((0,), (0,)), ((), ())), preferred_element_type=jnp.float32
    )

    @pl.when(i == 0)
    def _init():
        o_ref[...] = g

    @pl.when(i != 0)
    def _acc():
        o_ref[...] += g


def _make_main_body(n_edges: float):
    def _main_body(st_ref, w_ref, b_ref, gm_ref, bt_ref, s_ref, d_ref, a_ref,
                   o_ref):
        st = st_ref[...]
        Wm = w_ref[...]
        bb = b_ref[...]
        G = st[0:9, 0:9]
        csum = st[9:10, 0:9]
        cW = jnp.dot(csum, Wm, preferred_element_type=jnp.float32)
        GW = jnp.dot(G, Wm, preferred_element_type=jnp.float32)
        sumsq = (jnp.sum(Wm * GW, axis=0, keepdims=True)
                 + 2.0 * bb * cW + n_edges * bb * bb)
        mean = (cW + n_edges * bb) / n_edges
        var = sumsq / n_edges - mean * mean
        scale = gm_ref[...] * jax.lax.rsqrt(var + 1e-5)
        Wf = Wm * scale
        bf = (bb - mean) * scale + bt_ref[...]
        x = jnp.concatenate([s_ref[...], d_ref[...], a_ref[...]], axis=1)
        o_ref[...] = jnp.dot(x, Wf, preferred_element_type=jnp.float32) + bf

    return _main_body


def kernel(src, dest, edge_attr, W, b, gamma, beta):
    E = src.shape[0]
    BE = 3200
    nblk = E // BE

    stats = pl.pallas_call(
        _stats_body,
        grid=(nblk,),
        in_specs=[
            pl.BlockSpec((BE, 4), lambda i: (i, 0)),
            pl.BlockSpec((BE, 4), lambda i: (i, 0)),
            pl.BlockSpec((BE, 1), lambda i: (i, 0)),
        ],
        out_specs=pl.BlockSpec((10, 10), lambda i: (0, 0)),
        out_shape=jax.ShapeDtypeStruct((10, 10), jnp.float32),
    )(src, dest, edge_attr)

    b2 = b.reshape(1, 84)
    gm2 = gamma.reshape(1, 84)
    bt2 = beta.reshape(1, 84)

    out = pl.pallas_call(
        _make_main_body(float(E)),
        grid=(nblk,),
        in_specs=[
            pl.BlockSpec((10, 10), lambda i: (0, 0)),
            pl.BlockSpec((9, 84), lambda i: (0, 0)),
            pl.BlockSpec((1, 84), lambda i: (0, 0)),
            pl.BlockSpec((1, 84), lambda i: (0, 0)),
            pl.BlockSpec((1, 84), lambda i: (0, 0)),
            pl.BlockSpec((BE, 4), lambda i: (i, 0)),
            pl.BlockSpec((BE, 4), lambda i: (i, 0)),
            pl.BlockSpec((BE, 1), lambda i: (i, 0)),
        ],
        out_specs=pl.BlockSpec((BE, 84), lambda i: (i, 0)),
        out_shape=jax.ShapeDtypeStruct((E, 84), jnp.float32),
    )(stats, W, b2, gm2, bt2, src, dest, edge_attr)
    return out
